# baseline (speedup 1.0000x reference)
import jax
import jax.numpy as jnp
from jax import lax
from jax.experimental import pallas as pl

THRES_NMS = 0.7

B, Q, NOBJ, NVERB = 4, 100, 80, 117


def _hoi_kernel(obj_logits_ref, verb_logits_ref, sub_ref, obj_ref, ts_ref,
                cm_ref, hoi_ref, subout_ref, objout_ref, keep_ref):
    f32 = jnp.float32

    def iota2(n, dim):
        return lax.broadcasted_iota(jnp.int32, (n, n), dim)

    logits = obj_logits_ref[...].reshape(B * Q, NOBJ + 1)
    m = jnp.max(logits, axis=-1, keepdims=True)
    e = jnp.exp(logits - m)
    probs = e / jnp.sum(e, axis=-1, keepdims=True)
    p80 = probs[:, :NOBJ]
    obj_scores = jnp.max(p80, axis=-1, keepdims=True)
    eq = (p80 == obj_scores).astype(f32)
    lt80 = (iota2(NOBJ, 0) < iota2(NOBJ, 1)).astype(f32)
    cumb = lax.dot_general(eq, lt80, (((1,), (0,)), ((), ())),
                           preferred_element_type=f32)
    onehot = eq * (cumb == 0.0).astype(f32)

    verb = jax.nn.sigmoid(verb_logits_ref[...].reshape(B * Q, NVERB))
    cm80 = cm_ref[...][:, :NOBJ]
    masks = lax.dot_general(onehot, cm80, (((1,), (1,)), ((), ())),
                            preferred_element_type=f32)
    hoi = (verb * obj_scores) * masks
    hoi_ref[...] = hoi.reshape(B, Q, NVERB)

    ts = ts_ref[...].astype(f32)
    img_h = ts[:, 0:1]
    img_w = ts[:, 1:2]

    def to_xyxy(bt):
        cx, cy, w, h = bt[:, :, 0], bt[:, :, 1], bt[:, :, 2], bt[:, :, 3]
        x1 = (cx - 0.5 * w) * img_w
        y1 = (cy - 0.5 * h) * img_h
        x2 = (cx + 0.5 * w) * img_w
        y2 = (cy + 0.5 * h) * img_h
        return x1, y1, x2, y2

    sx1, sy1, sx2, sy2 = to_xyxy(sub_ref[...])
    ox1, oy1, ox2, oy2 = to_xyxy(obj_ref[...])
    subout_ref[...] = jnp.stack([sx1, sy1, sx2, sy2], axis=-1)
    objout_ref[...] = jnp.stack([ox1, oy1, ox2, oy2], axis=-1)

    s_area = (sx2 - sx1 + 1) * (sy2 - sy1 + 1)
    o_area = (ox2 - ox1 + 1) * (oy2 - oy1 + 1)

    def pair_iou(x1, y1, x2, y2, area):
        xx1 = jnp.maximum(x1[:, :, None], x1[:, None, :])
        yy1 = jnp.maximum(y1[:, :, None], y1[:, None, :])
        xx2 = jnp.minimum(x2[:, :, None], x2[:, None, :])
        yy2 = jnp.minimum(y2[:, :, None], y2[:, None, :])
        w = jnp.maximum(0.0, xx2 - xx1 + 1)
        h = jnp.maximum(0.0, yy2 - yy1 + 1)
        inter = w * h
        union = area[:, :, None] + area[:, None, :] - inter
        return inter / union

    s_iou = pair_iou(sx1, sy1, sx2, sy2, s_area)
    o_iou = pair_iou(ox1, oy1, ox2, oy2, o_area)
    ovr = s_iou * jnp.sqrt(o_iou)

    oh3 = onehot.reshape(B, Q, NOBJ)
    same = lax.dot_general(oh3, oh3, (((2,), (2,)), ((0,), (0,))),
                           preferred_element_type=f32)
    noteye = 1.0 - (iota2(Q, 0) == iota2(Q, 1)).astype(f32)
    thres = (ovr > THRES_NMS).astype(f32)
    Mmat = same * thres * noteye

    ms_sub = jnp.max(hoi.reshape(B, Q, NVERB), axis=-1, keepdims=True)
    ms_lane = jnp.transpose(ms_sub, (0, 2, 1))
    lt2 = (iota2(Q, 0) < iota2(Q, 1)).astype(f32)
    gt = (ms_sub > ms_lane).astype(f32)
    tie = (ms_sub == ms_lane).astype(f32) * lt2
    rank_lane = jnp.sum(gt + tie, axis=1, keepdims=True)
    iq_sub = iota2(Q, 0).astype(f32)
    P = (rank_lane == iq_sub).astype(f32)

    tmp = lax.dot_general(P, Mmat, (((2,), (1,)), ((0,), (0,))),
                          preferred_element_type=f32)
    Ms = lax.dot_general(tmp, P, (((2,), (2,)), ((0,), (0,))),
                         preferred_element_type=f32)

    MsU = Ms * lt2
    a0 = rank_lane * 0.0 + 1.0

    def step(cur):
        t = lax.dot_general(cur, MsU, (((2,), (1,)), ((0,), (0,))),
                            preferred_element_type=f32)
        return (t == 0.0).astype(f32)

    def wcond(c):
        old, new = c
        return jnp.sum(jnp.abs(new - old)) > 0.0

    def wbody(c):
        _, cur = c
        return (cur, step(cur))

    _, alive = lax.while_loop(wcond, wbody, (a0, step(a0)))

    keep3 = lax.dot_general(alive, P, (((2,), (1,)), ((0,), (0,))),
                            preferred_element_type=f32)
    keep_ref[...] = keep3[:, 0, :]


@jax.jit
def kernel(pred_obj_logits, pred_verb_logits, pred_sub_boxes, pred_obj_boxes,
           target_sizes, correct_mat):
    out_shapes = (
        jax.ShapeDtypeStruct((B, Q, NVERB), jnp.float32),
        jax.ShapeDtypeStruct((B, Q, 4), jnp.float32),
        jax.ShapeDtypeStruct((B, Q, 4), jnp.float32),
        jax.ShapeDtypeStruct((B, Q), jnp.float32),
    )
    return pl.pallas_call(
        _hoi_kernel,
        out_shape=out_shapes,
    )(pred_obj_logits, pred_verb_logits, pred_sub_boxes, pred_obj_boxes,
      target_sizes, correct_mat)


# speedup vs baseline: 1.3590x; 1.3590x over previous
import jax
import jax.numpy as jnp
from jax import lax
from jax.experimental import pallas as pl

THRES_NMS = 0.7

B, Q, NOBJ, NVERB = 4, 100, 80, 117


def _hoi_kernel(obj_logits_ref, verb_logits_ref, subT_ref, objT_ref, ts_ref,
                cm_ref, hoi_ref, subout_ref, objout_ref, keep_ref):
    f32 = jnp.float32

    def iota2(n, dim):
        return lax.broadcasted_iota(jnp.int32, (n, n), dim)

    logits = obj_logits_ref[...].reshape(B * Q, NOBJ + 1)
    m = jnp.max(logits, axis=-1, keepdims=True)
    e = jnp.exp(logits - m)
    probs = e / jnp.sum(e, axis=-1, keepdims=True)
    p80 = probs[:, :NOBJ]
    obj_scores = jnp.max(p80, axis=-1, keepdims=True)
    eq = (p80 == obj_scores).astype(f32)
    lt80 = (iota2(NOBJ, 0) < iota2(NOBJ, 1)).astype(f32)
    cumb = lax.dot_general(eq, lt80, (((1,), (0,)), ((), ())),
                           preferred_element_type=f32)
    onehot = eq * (cumb == 0.0).astype(f32)

    verb = jax.nn.sigmoid(verb_logits_ref[...].reshape(B * Q, NVERB))
    cm80 = cm_ref[...][:, :NOBJ]
    masks = lax.dot_general(onehot, cm80, (((1,), (1,)), ((), ())),
                            preferred_element_type=f32)
    hoi = (verb * obj_scores) * masks
    hoi_ref[...] = hoi.reshape(B, Q, NVERB)

    ts = ts_ref[...]
    img_h = ts[:, 0:1]
    img_w = ts[:, 1:2]

    def to_xyxy(bt):
        cx, cy, w, h = bt[:, 0, :], bt[:, 1, :], bt[:, 2, :], bt[:, 3, :]
        x1 = (cx - 0.5 * w) * img_w
        y1 = (cy - 0.5 * h) * img_h
        x2 = (cx + 0.5 * w) * img_w
        y2 = (cy + 0.5 * h) * img_h
        return x1, y1, x2, y2

    sx1, sy1, sx2, sy2 = to_xyxy(subT_ref[...])
    ox1, oy1, ox2, oy2 = to_xyxy(objT_ref[...])
    subout_ref[...] = jnp.stack([sx1, sy1, sx2, sy2], axis=1)
    objout_ref[...] = jnp.stack([ox1, oy1, ox2, oy2], axis=1)

    s_area = (sx2 - sx1 + 1) * (sy2 - sy1 + 1)
    o_area = (ox2 - ox1 + 1) * (oy2 - oy1 + 1)

    def pair_iou(x1, y1, x2, y2, area):
        xx1 = jnp.maximum(x1[:, :, None], x1[:, None, :])
        yy1 = jnp.maximum(y1[:, :, None], y1[:, None, :])
        xx2 = jnp.minimum(x2[:, :, None], x2[:, None, :])
        yy2 = jnp.minimum(y2[:, :, None], y2[:, None, :])
        w = jnp.maximum(0.0, xx2 - xx1 + 1)
        h = jnp.maximum(0.0, yy2 - yy1 + 1)
        inter = w * h
        union = area[:, :, None] + area[:, None, :] - inter
        return inter / union

    s_iou = pair_iou(sx1, sy1, sx2, sy2, s_area)
    o_iou = pair_iou(ox1, oy1, ox2, oy2, o_area)
    ovr = s_iou * jnp.sqrt(o_iou)

    oh3 = onehot.reshape(B, Q, NOBJ)
    same = lax.dot_general(oh3, oh3, (((2,), (2,)), ((0,), (0,))),
                           preferred_element_type=f32)
    noteye = 1.0 - (iota2(Q, 0) == iota2(Q, 1)).astype(f32)
    thres = (ovr > THRES_NMS).astype(f32)
    Mmat = same * thres * noteye

    ms_sub = jnp.max(hoi.reshape(B, Q, NVERB), axis=-1, keepdims=True)
    ms_lane = jnp.transpose(ms_sub, (0, 2, 1))
    lt2 = (iota2(Q, 0) < iota2(Q, 1)).astype(f32)
    gt = (ms_sub > ms_lane).astype(f32)
    tie = (ms_sub == ms_lane).astype(f32) * lt2
    rank_lane = jnp.sum(gt + tie, axis=1, keepdims=True)
    iq_sub = iota2(Q, 0).astype(f32)
    P = (rank_lane == iq_sub).astype(f32)

    tmp = lax.dot_general(P, Mmat, (((2,), (1,)), ((0,), (0,))),
                          preferred_element_type=f32)
    Ms = lax.dot_general(tmp, P, (((2,), (2,)), ((0,), (0,))),
                         preferred_element_type=f32)

    MsU = Ms * lt2
    a0 = rank_lane * 0.0 + 1.0

    def step(cur):
        t = lax.dot_general(cur, MsU, (((2,), (1,)), ((0,), (0,))),
                            preferred_element_type=f32)
        return (t == 0.0).astype(f32)

    def wcond(c):
        old, new = c
        return jnp.sum(jnp.abs(new - old)) > 0.0

    def wbody(c):
        _, cur = c
        return (cur, step(cur))

    _, alive = lax.while_loop(wcond, wbody, (a0, step(a0)))

    keep3 = lax.dot_general(alive, P, (((2,), (1,)), ((0,), (0,))),
                            preferred_element_type=f32)
    keep_ref[...] = keep3[:, 0, :]


@jax.jit
def kernel(pred_obj_logits, pred_verb_logits, pred_sub_boxes, pred_obj_boxes,
           target_sizes, correct_mat):
    subT = jnp.transpose(pred_sub_boxes, (0, 2, 1))
    objT = jnp.transpose(pred_obj_boxes, (0, 2, 1))
    ts_f = target_sizes.astype(jnp.float32)

    out_shapes = (
        jax.ShapeDtypeStruct((B, Q, NVERB), jnp.float32),
        jax.ShapeDtypeStruct((B, 4, Q), jnp.float32),
        jax.ShapeDtypeStruct((B, 4, Q), jnp.float32),
        jax.ShapeDtypeStruct((B, Q), jnp.float32),
    )
    hoi, subT_o, objT_o, keep = pl.pallas_call(
        _hoi_kernel,
        out_shape=out_shapes,
    )(pred_obj_logits, pred_verb_logits, subT, objT, ts_f, correct_mat)

    sub_boxes = jnp.transpose(subT_o, (0, 2, 1))
    obj_boxes = jnp.transpose(objT_o, (0, 2, 1))
    return (hoi, sub_boxes, obj_boxes, keep)
